# baseline (speedup 1.0000x reference)
import functools

import jax
import jax.numpy as jnp
from jax import lax
from jax.experimental import pallas as pl
from jax.experimental.pallas import tpu as pltpu
from jax.experimental.pallas import tpu_sc as plsc

D_MODEL = 768
FFN = 1024
E = 64
BT = 64


def _router_body(x_ref, rw_ref, pos_ref, starts_ref, counts_ref, xb_ref):
    n = x_ref.shape[0]
    x = x_ref[...]
    rw = rw_ref[...]
    logits = lax.dot_general(x, rw, (((1,), (1,)), ((), ())),
                             preferred_element_type=jnp.float32)
    ids = lax.broadcasted_iota(jnp.int32, (n, E), 1)
    m = jnp.max(logits, axis=1, keepdims=True)
    eid = jnp.min(jnp.where(logits == m, ids, E), axis=1, keepdims=True)
    P = (ids == eid).astype(jnp.float32)
    nb = n // 128
    P3 = P.reshape(nb, 128, E)
    S = jnp.sum(P3, axis=1)
    counts = jnp.sum(S, axis=0)
    ui = lax.broadcasted_iota(jnp.int32, (E, E), 0)
    uj = lax.broadcasted_iota(jnp.int32, (E, E), 1)
    U = (ui < uj).astype(jnp.float32)
    offs = lax.dot_general(counts.reshape(1, E), U, (((1,), (0,)), ((), ())),
                           preferred_element_type=jnp.float32)
    bi = lax.broadcasted_iota(jnp.int32, (nb, nb), 0)
    bj = lax.broadcasted_iota(jnp.int32, (nb, nb), 1)
    Lb = (bj < bi).astype(jnp.float32)
    prior = lax.dot_general(Lb, S, (((1,), (0,)), ((), ())),
                            preferred_element_type=jnp.float32)
    li = lax.broadcasted_iota(jnp.int32, (128, 128), 0)
    lj = lax.broadcasted_iota(jnp.int32, (128, 128), 1)
    L128 = jnp.broadcast_to(((lj <= li).astype(jnp.float32))[None],
                            (nb, 128, 128))
    C3 = lax.dot_general(L128, P3, (((2,), (1,)), ((0,), (0,))),
                         preferred_element_type=jnp.float32)
    tot = C3 + prior[:, None, :] + offs.reshape(1, 1, E)
    pos_f = jnp.sum(P3 * tot, axis=2) - 1.0
    pos_ref[...] = pos_f.reshape(1, n).astype(jnp.int32)
    starts_ref[...] = offs.reshape(E).astype(jnp.int32)
    counts_ref[...] = counts.astype(jnp.int32)
    xb_ref[...] = x.astype(jnp.bfloat16)


def _route(xf, router_w):
    n = xf.shape[0]
    return pl.pallas_call(
        _router_body,
        out_shape=(
            jax.ShapeDtypeStruct((1, n), jnp.int32),
            jax.ShapeDtypeStruct((E,), jnp.int32),
            jax.ShapeDtypeStruct((E,), jnp.int32),
            jax.ShapeDtypeStruct((n, xf.shape[1]), jnp.bfloat16),
        ),
        compiler_params=pltpu.CompilerParams(
            vmem_limit_bytes=100 * 1024 * 1024),
    )(xf, router_w)


def _make_sc_permute(n, d, gather):
    info = plsc.get_sparse_core_info()
    nc, ns = info.num_cores, info.num_subcores
    nw = nc * ns
    assert n % (8 * nw) == 0
    chunk = n // nw
    mesh = plsc.VectorSubcoreMesh(core_axis_name="c", subcore_axis_name="s")

    half = chunk // 2

    @functools.partial(
        pl.kernel,
        out_type=jax.ShapeDtypeStruct((n, d), jnp.float32),
        mesh=mesh,
        scratch_types=[
            pltpu.VMEM((half,), jnp.int32),
            pltpu.VMEM((half,), jnp.int32),
            pltpu.VMEM((half, d), jnp.float32),
            pltpu.VMEM((half, d), jnp.float32),
            pltpu.SemaphoreType.DMA,
            pltpu.SemaphoreType.DMA,
            pltpu.SemaphoreType.DMA,
            pltpu.SemaphoreType.DMA,
        ],
    )
    def k(rows_hbm, pos_hbm, out_hbm, idx0, idx1, buf0, buf1,
          si0, si1, s0, s1):
        wid = lax.axis_index("s") * nc + lax.axis_index("c")
        base = wid * chunk
        ci0 = pltpu.async_copy(pos_hbm.at[pl.ds(base, half)], idx0, si0)
        ci1 = pltpu.async_copy(pos_hbm.at[pl.ds(base + half, half)], idx1, si1)
        if gather:
            ci0.wait()
            g0 = pltpu.async_copy(rows_hbm.at[idx0], buf0, s0)
            ci1.wait()
            g1 = pltpu.async_copy(rows_hbm.at[idx1], buf1, s1)
            g0.wait()
            w0 = pltpu.async_copy(buf0, out_hbm.at[pl.ds(base, half)], si0)
            g1.wait()
            w1 = pltpu.async_copy(buf1, out_hbm.at[pl.ds(base + half, half)],
                                  si1)
            w0.wait()
            w1.wait()
        else:
            l0 = pltpu.async_copy(rows_hbm.at[pl.ds(base, half)], buf0, s0)
            l1 = pltpu.async_copy(rows_hbm.at[pl.ds(base + half, half)],
                                  buf1, s1)
            ci0.wait()
            l0.wait()
            w0 = pltpu.async_copy(buf0, out_hbm.at[idx0], si0)
            ci1.wait()
            l1.wait()
            w1 = pltpu.async_copy(buf1, out_hbm.at[idx1], si1)
            w0.wait()
            w1.wait()

    return k


EPB = 2


def _expert_body(starts_ref, counts_ref, pos_ref, xb_ref, w1_ref, v1_ref,
                 w2_ref, out_ref):
    n = xb_ref.shape[0]
    g0 = pl.program_id(0)

    @pl.when(g0 == 0)
    def _init():
        out_ref[...] = jnp.zeros_like(out_ref)

    posr = pos_ref[...]

    for sub in range(EPB):
        e = g0 * EPB + sub
        start = starts_ref[e]
        cnt = counts_ref[e]
        w1 = w1_ref[sub].astype(jnp.bfloat16)
        v1 = v1_ref[sub].astype(jnp.bfloat16)
        w2 = w2_ref[sub].astype(jnp.bfloat16)
        a0 = (start // 8) * 8
        end = start + cnt
        nt = (end - a0 + BT - 1) // BT

        def body(j, _, start=start, end=end, a0=a0, w1=w1, v1=v1, w2=w2):
            row0 = a0 + j * BT
            c0 = pl.multiple_of(jnp.minimum(row0, n - BT), 8)
            sl = c0 + lax.broadcasted_iota(jnp.int32, (BT, 1), 0)
            Qe = (posr == sl).astype(jnp.bfloat16)
            xt = lax.dot_general(Qe, xb_ref[...], (((1,), (0,)), ((), ())),
                                 preferred_element_type=jnp.float32
                                 ).astype(jnp.bfloat16)
            g = lax.dot_general(xt, w1, (((1,), (1,)), ((), ())),
                                preferred_element_type=jnp.float32)
            u = lax.dot_general(xt, v1, (((1,), (1,)), ((), ())),
                                preferred_element_type=jnp.float32)
            h = g * jax.nn.sigmoid(g) * u
            o = lax.dot_general(h.astype(jnp.bfloat16), w2,
                                (((1,), (1,)), ((), ())),
                                preferred_element_type=jnp.float32)
            mask = (sl >= jnp.maximum(row0, start)) & (sl < end)
            out_ref[pl.ds(c0, BT), :] += jnp.where(mask, o, 0.0)
            return 0

        lax.fori_loop(0, nt, body, 0)


def _experts(starts, counts, pos, xb, w1, v1, w2):
    n = xb.shape[0]
    return pl.pallas_call(
        _expert_body,
        grid=(E // EPB,),
        in_specs=[
            pl.BlockSpec(memory_space=pltpu.SMEM),
            pl.BlockSpec(memory_space=pltpu.SMEM),
            pl.BlockSpec((1, n), lambda e: (0, 0)),
            pl.BlockSpec((n, D_MODEL), lambda e: (0, 0)),
            pl.BlockSpec((EPB, FFN, D_MODEL), lambda e: (e, 0, 0)),
            pl.BlockSpec((EPB, FFN, D_MODEL), lambda e: (e, 0, 0)),
            pl.BlockSpec((EPB, D_MODEL, FFN), lambda e: (e, 0, 0)),
        ],
        out_specs=pl.BlockSpec((n, D_MODEL), lambda e: (0, 0)),
        out_shape=jax.ShapeDtypeStruct((n, D_MODEL), jnp.float32),
        compiler_params=pltpu.CompilerParams(
            dimension_semantics=("arbitrary",),
            vmem_limit_bytes=100 * 1024 * 1024),
    )(starts, counts, pos, xb, w1, v1, w2)


def kernel(x, router_w, w1, v1, w2):
    bsz, q_len, d = x.shape
    xf = x.reshape(-1, d)
    n = xf.shape[0]
    pos, starts, counts, xb = _route(xf, router_w)
    out_sorted = _experts(starts, counts, pos, xb, w1, v1, w2)
    out = _make_sc_permute(n, d, gather=True)(out_sorted, pos.reshape(n))
    return out.reshape(bsz, q_len, d)


# speedup vs baseline: 1.0520x; 1.0520x over previous
import functools

import jax
import jax.numpy as jnp
from jax import lax
from jax.experimental import pallas as pl
from jax.experimental.pallas import tpu as pltpu
from jax.experimental.pallas import tpu_sc as plsc

D_MODEL = 768
FFN = 1024
E = 64
BT = 64


def _router_body(x_ref, rw_ref, pos_ref, starts_ref, counts_ref):
    n = x_ref.shape[0]
    x = x_ref[...]
    rw = rw_ref[...]
    logits = lax.dot_general(x, rw, (((1,), (1,)), ((), ())),
                             preferred_element_type=jnp.float32)
    ids = lax.broadcasted_iota(jnp.int32, (n, E), 1)
    m = jnp.max(logits, axis=1, keepdims=True)
    eid = jnp.min(jnp.where(logits == m, ids, E), axis=1, keepdims=True)
    P = (ids == eid).astype(jnp.float32)
    nb = n // 128
    P3 = P.reshape(nb, 128, E)
    S = jnp.sum(P3, axis=1)
    counts = jnp.sum(S, axis=0)
    ui = lax.broadcasted_iota(jnp.int32, (E, E), 0)
    uj = lax.broadcasted_iota(jnp.int32, (E, E), 1)
    U = (ui < uj).astype(jnp.float32)
    offs = lax.dot_general(counts.reshape(1, E), U, (((1,), (0,)), ((), ())),
                           preferred_element_type=jnp.float32)
    bi = lax.broadcasted_iota(jnp.int32, (nb, nb), 0)
    bj = lax.broadcasted_iota(jnp.int32, (nb, nb), 1)
    Lb = (bj < bi).astype(jnp.float32)
    prior = lax.dot_general(Lb, S, (((1,), (0,)), ((), ())),
                            preferred_element_type=jnp.float32)
    li = lax.broadcasted_iota(jnp.int32, (128, 128), 0)
    lj = lax.broadcasted_iota(jnp.int32, (128, 128), 1)
    L128 = jnp.broadcast_to(((lj <= li).astype(jnp.float32))[None],
                            (nb, 128, 128))
    C3 = lax.dot_general(L128, P3, (((2,), (1,)), ((0,), (0,))),
                         preferred_element_type=jnp.float32)
    tot = C3 + prior[:, None, :] + offs.reshape(1, 1, E)
    pos_f = jnp.sum(P3 * tot, axis=2) - 1.0
    pos_ref[...] = pos_f.reshape(n).astype(jnp.int32)
    starts_ref[...] = offs.reshape(E).astype(jnp.int32)
    counts_ref[...] = counts.astype(jnp.int32)


def _route(xf, router_w):
    n = xf.shape[0]
    return pl.pallas_call(
        _router_body,
        out_shape=(
            jax.ShapeDtypeStruct((n,), jnp.int32),
            jax.ShapeDtypeStruct((E,), jnp.int32),
            jax.ShapeDtypeStruct((E,), jnp.int32),
        ),
        compiler_params=pltpu.CompilerParams(
            vmem_limit_bytes=100 * 1024 * 1024),
    )(xf, router_w)


def _make_sc_permute(n, d, gather):
    info = plsc.get_sparse_core_info()
    nc, ns = info.num_cores, info.num_subcores
    nw = nc * ns
    assert n % (8 * nw) == 0
    chunk = n // nw
    mesh = plsc.VectorSubcoreMesh(core_axis_name="c", subcore_axis_name="s")

    half = chunk // 2

    @functools.partial(
        pl.kernel,
        out_type=jax.ShapeDtypeStruct((n, d), jnp.float32),
        mesh=mesh,
        scratch_types=[
            pltpu.VMEM((half,), jnp.int32),
            pltpu.VMEM((half,), jnp.int32),
            pltpu.VMEM((half, d), jnp.float32),
            pltpu.VMEM((half, d), jnp.float32),
            pltpu.SemaphoreType.DMA,
            pltpu.SemaphoreType.DMA,
            pltpu.SemaphoreType.DMA,
            pltpu.SemaphoreType.DMA,
        ],
    )
    def k(rows_hbm, pos_hbm, out_hbm, idx0, idx1, buf0, buf1,
          si0, si1, s0, s1):
        wid = lax.axis_index("s") * nc + lax.axis_index("c")
        base = wid * chunk
        ci0 = pltpu.async_copy(pos_hbm.at[pl.ds(base, half)], idx0, si0)
        ci1 = pltpu.async_copy(pos_hbm.at[pl.ds(base + half, half)], idx1, si1)
        if gather:
            ci0.wait()
            g0 = pltpu.async_copy(rows_hbm.at[idx0], buf0, s0)
            ci1.wait()
            g1 = pltpu.async_copy(rows_hbm.at[idx1], buf1, s1)
            g0.wait()
            w0 = pltpu.async_copy(buf0, out_hbm.at[pl.ds(base, half)], si0)
            g1.wait()
            w1 = pltpu.async_copy(buf1, out_hbm.at[pl.ds(base + half, half)],
                                  si1)
            w0.wait()
            w1.wait()
        else:
            l0 = pltpu.async_copy(rows_hbm.at[pl.ds(base, half)], buf0, s0)
            l1 = pltpu.async_copy(rows_hbm.at[pl.ds(base + half, half)],
                                  buf1, s1)
            ci0.wait()
            l0.wait()
            w0 = pltpu.async_copy(buf0, out_hbm.at[idx0], si0)
            ci1.wait()
            l1.wait()
            w1 = pltpu.async_copy(buf1, out_hbm.at[idx1], si1)
            w0.wait()
            w1.wait()

    return k


EPB = 2


def _expert_body(starts_ref, counts_ref, xs_ref, w1_ref, v1_ref,
                 w2_ref, out_ref):
    n = xs_ref.shape[0]
    g0 = pl.program_id(0)

    @pl.when(g0 == 0)
    def _init():
        out_ref[...] = jnp.zeros_like(out_ref)

    for sub in range(EPB):
        e = g0 * EPB + sub
        start = starts_ref[e]
        cnt = counts_ref[e]
        w1 = w1_ref[sub].astype(jnp.bfloat16)
        v1 = v1_ref[sub].astype(jnp.bfloat16)
        w2 = w2_ref[sub].astype(jnp.bfloat16)
        a0 = (start // 8) * 8
        end = start + cnt
        nt = (end - a0 + BT - 1) // BT

        def body(j, _, start=start, end=end, a0=a0, w1=w1, v1=v1, w2=w2):
            row0 = a0 + j * BT
            c0 = pl.multiple_of(jnp.minimum(row0, n - BT), 8)
            xt = xs_ref[pl.ds(c0, BT), :].astype(jnp.bfloat16)
            g = lax.dot_general(xt, w1, (((1,), (1,)), ((), ())),
                                preferred_element_type=jnp.float32)
            u = lax.dot_general(xt, v1, (((1,), (1,)), ((), ())),
                                preferred_element_type=jnp.float32)
            h = g * jax.nn.sigmoid(g) * u
            o = lax.dot_general(h.astype(jnp.bfloat16), w2,
                                (((1,), (1,)), ((), ())),
                                preferred_element_type=jnp.float32)
            gidx = c0 + lax.broadcasted_iota(jnp.int32, (BT, 1), 0)
            mask = (gidx >= jnp.maximum(row0, start)) & (gidx < end)
            out_ref[pl.ds(c0, BT), :] += jnp.where(mask, o, 0.0)
            return 0

        lax.fori_loop(0, nt, body, 0)


def _experts(starts, counts, x_sorted, w1, v1, w2):
    n = x_sorted.shape[0]
    return pl.pallas_call(
        _expert_body,
        grid=(E // EPB,),
        in_specs=[
            pl.BlockSpec(memory_space=pltpu.SMEM),
            pl.BlockSpec(memory_space=pltpu.SMEM),
            pl.BlockSpec((n, D_MODEL), lambda e: (0, 0)),
            pl.BlockSpec((EPB, FFN, D_MODEL), lambda e: (e, 0, 0)),
            pl.BlockSpec((EPB, FFN, D_MODEL), lambda e: (e, 0, 0)),
            pl.BlockSpec((EPB, D_MODEL, FFN), lambda e: (e, 0, 0)),
        ],
        out_specs=pl.BlockSpec((n, D_MODEL), lambda e: (0, 0)),
        out_shape=jax.ShapeDtypeStruct((n, D_MODEL), jnp.float32),
        compiler_params=pltpu.CompilerParams(
            dimension_semantics=("arbitrary",),
            vmem_limit_bytes=100 * 1024 * 1024),
    )(starts, counts, x_sorted, w1, v1, w2)


def kernel(x, router_w, w1, v1, w2):
    bsz, q_len, d = x.shape
    xf = x.reshape(-1, d)
    n = xf.shape[0]
    pos, starts, counts = _route(xf, router_w)
    x_sorted = _make_sc_permute(n, d, gather=False)(xf, pos)
    out_sorted = _experts(starts, counts, x_sorted, w1, v1, w2)
    out = _make_sc_permute(n, d, gather=True)(out_sorted, pos)
    return out.reshape(bsz, q_len, d)
